# baseline (speedup 1.0000x reference)
import functools

import jax
import jax.numpy as jnp
from jax import lax
from jax.experimental import pallas as pl
from jax.experimental.pallas import tpu as pltpu
from jax.experimental.pallas import tpu_sc as plsc

N = 10000
E = 320000
D = 128
FFD = 256
NC = 2
NS = 16
NW = NC * NS
EPT = E // NW
C = 80
NCH = EPT // C
W = 25
NWIN = NCH // W
NPAD = 10240

_sc_mesh = plsc.VectorSubcoreMesh(
    core_axis_name="c", subcore_axis_name="s", num_cores=NC, num_subcores=NS)


@functools.partial(
    pl.kernel,
    out_type=jax.ShapeDtypeStruct((NC, NPAD), jnp.float32),
    mesh=_sc_mesh,
    scratch_types=[
        pltpu.VMEM((NCH, C), jnp.int32),
        pltpu.VMEM((C,), jnp.float32),
        pltpu.VMEM((NPAD // NS,), jnp.float32),
        pltpu.VMEM_SHARED((NPAD,), jnp.float32),
        pltpu.SemaphoreType.DMA,
    ],
)
def _deg_kernel(e4_hbm, deg_out, dst_v, ones_v, z_v, deg_sh, isem):
    cid = lax.axis_index("c")
    sid = lax.axis_index("s")
    wid = sid * NC + cid

    for w in range(NWIN):
        pltpu.async_copy(e4_hbm.at[1, wid, w],
                         dst_v.at[pl.ds(w * W, W)], isem)

    def zb(i, carry):
        z_v[pl.ds(i * 16, 16)] = jnp.zeros((16,), jnp.float32)
        return carry

    lax.fori_loop(0, NPAD // NS // 16, zb, 0)
    pltpu.sync_copy(z_v, deg_sh.at[pl.ds(sid * (NPAD // NS), NPAD // NS)])
    for k in range(C // 16):
        ones_v[pl.ds(k * 16, 16)] = jnp.full((16,), 1.0, jnp.float32)
    for w in range(NWIN):
        pltpu.make_async_copy(e4_hbm.at[1, wid, w],
                              dst_v.at[pl.ds(w * W, W)], isem).wait()
    plsc.subcore_barrier()

    def body(j, carry):
        pltpu.sync_copy(ones_v, deg_sh.at[dst_v.at[j]], add=True)
        return carry

    lax.fori_loop(0, NCH, body, 0)
    plsc.subcore_barrier()

    @pl.when(sid == 0)
    def _():
        pltpu.sync_copy(deg_sh, deg_out.at[cid])


def _make_gs(mode):
  @functools.partial(
    pl.kernel,
    out_type=jax.ShapeDtypeStruct((NC, N, D), jnp.float32),
    mesh=_sc_mesh,
    scratch_types=[
        pltpu.VMEM((2, W, C), jnp.int32),
        pltpu.VMEM((2, W, C), jnp.int32),
        pltpu.VMEM((3, C, D), jnp.float32),
        pltpu.VMEM_SHARED((N, D), jnp.float32),
        pltpu.SemaphoreType.DMA,
        pltpu.SemaphoreType.DMA,
        pltpu.SemaphoreType.DMA,
    ],
  )
  def _gs_kernel(y_hbm, e4_hbm, agg_out,
                 src_v, dst_v, rows_v, agg_sh, gsem, ssem, isem):
    cid = lax.axis_index("c")
    sid = lax.axis_index("s")
    wid = sid * NC + cid

    def zbody(r, carry):
        for k in range(D // 16):
            rows_v[0, r, pl.ds(k * 16, 16)] = jnp.zeros((16,), jnp.float32)
        return carry

    lax.fori_loop(0, C, zbody, 0)
    for k in range(8):
        ch = sid + k * NS

        @pl.when(ch < NCH)
        def _():
            pltpu.sync_copy(rows_v.at[0], agg_sh.at[pl.ds(ch * C, C)])

    plsc.subcore_barrier()

    def _drain_scatter():
        pltpu.make_async_copy(rows_v.at[0], agg_sh.at[dst_v.at[0, 0]],
                              ssem).wait()

    pltpu.async_copy(e4_hbm.at[0, wid, 0], src_v.at[0], isem)
    pltpu.async_copy(e4_hbm.at[1, wid, 0], dst_v.at[0], isem)

    def wbody(w, carry):
        wb = lax.rem(w, 2)
        pltpu.make_async_copy(e4_hbm.at[0, wid, w], src_v.at[wb],
                              isem).wait()
        pltpu.make_async_copy(e4_hbm.at[1, wid, w], dst_v.at[wb],
                              isem).wait()
        if mode != "scatter":
            pltpu.async_copy(y_hbm.at[src_v.at[wb, 0]], rows_v.at[0], gsem)
        @pl.when(w + 1 < NWIN)
        def _():
            nwb = lax.rem(w + 1, 2)
            pltpu.async_copy(e4_hbm.at[0, wid, w + 1], src_v.at[nwb], isem)
            pltpu.async_copy(e4_hbm.at[1, wid, w + 1], dst_v.at[nwb], isem)

        def body(j, carry2):
            if mode != "gather":
                @pl.when(j >= 2)
                def _():
                    _drain_scatter()

            if mode != "scatter":
                @pl.when(j + 1 < W)
                def _():
                    pltpu.async_copy(y_hbm.at[src_v.at[wb, j + 1]],
                                     rows_v.at[lax.rem(j + 1, 3)], gsem)

                pltpu.make_async_copy(y_hbm.at[src_v.at[wb, j]],
                                      rows_v.at[lax.rem(j, 3)], gsem).wait()
            if mode != "gather":
                pltpu.async_copy(rows_v.at[lax.rem(j, 3)],
                                 agg_sh.at[dst_v.at[wb, j]], ssem, add=True)
            return carry2

        lax.fori_loop(0, W, body, 0)
        if mode != "gather":
            _drain_scatter()
            _drain_scatter()
        return carry

    lax.fori_loop(0, NWIN, wbody, 0)
    plsc.subcore_barrier()

    for k in range(8):
        ch = sid + k * NS

        @pl.when(ch < NCH)
        def _():
            pltpu.sync_copy(agg_sh.at[pl.ds(ch * C, C)],
                            agg_out.at[cid, pl.ds(ch * C, C)])

  return _gs_kernel


_gs_kernel = _make_gs("full")


_BM = 1000
_G = N // _BM


def _ln(h):
    mu = jnp.mean(h, axis=-1, keepdims=True)
    var = jnp.mean((h - mu) ** 2, axis=-1, keepdims=True)
    return (h - mu) * lax.rsqrt(var + 1e-5)


def _prep_body(x_ref, da_ref, db_ref, dinv_ref, y_ref):
    deg = da_ref[...] + db_ref[...] + 1.0
    dinv = lax.rsqrt(deg)
    dinv_ref[...] = dinv
    y_ref[...] = x_ref[...] * dinv


_prep = pl.pallas_call(
    _prep_body,
    grid=(_G,),
    in_specs=[
        pl.BlockSpec((_BM, D), lambda i: (i, 0)),
        pl.BlockSpec((_BM, 1), lambda i: (i, 0)),
        pl.BlockSpec((_BM, 1), lambda i: (i, 0)),
    ],
    out_specs=[
        pl.BlockSpec((_BM, 1), lambda i: (i, 0)),
        pl.BlockSpec((_BM, D), lambda i: (i, 0)),
    ],
    out_shape=[
        jax.ShapeDtypeStruct((N, 1), jnp.float32),
        jax.ShapeDtypeStruct((N, D), jnp.float32),
    ],
)


def _dense_chain(xb, dv, p0, p1, wg, bg, w1, b1, w2, b2, l1g, l1b, l2g, l2b):
    agg = dv * (p0 + p1) + (dv * dv) * xb
    hg = jnp.dot(agg, wg, preferred_element_type=jnp.float32)
    h = xb + hg + bg
    h = _ln(h) * l1g + l1b
    t = jnp.maximum(
        jnp.dot(h, w1, preferred_element_type=jnp.float32) + b1, 0.0)
    ff = jnp.dot(t, w2, preferred_element_type=jnp.float32) + b2
    return _ln(h + ff) * l2g + l2b


def _layer0_body(x_ref, p0_ref, p1_ref, dinv_ref,
                 wg_ref, bg_ref, w1_ref, b1_ref, w2_ref, b2_ref,
                 l1g_ref, l1b_ref, l2g_ref, l2b_ref, lvw_ref, lvb_ref,
                 xo_ref, yo_ref, pado_ref):
    dv = dinv_ref[...]
    h2 = _dense_chain(x_ref[...], dv, p0_ref[0], p1_ref[0],
                      wg_ref[...], bg_ref[...], w1_ref[...], b1_ref[...],
                      w2_ref[...], b2_ref[...], l1g_ref[...], l1b_ref[...],
                      l2g_ref[...], l2b_ref[...])
    xo_ref[...] = h2
    yo_ref[...] = h2 * dv
    pado_ref[...] = jnp.dot(
        h2, lvw_ref[...], preferred_element_type=jnp.float32) + lvb_ref[...]


def _layerf_body(x_ref, p0_ref, p1_ref, dinv_ref, pad_ref,
                 wg_ref, bg_ref, w1_ref, b1_ref, w2_ref, b2_ref,
                 l1g_ref, l1b_ref, l2g_ref, l2b_ref, lvw_ref, lvb_ref,
                 out_ref):
    dv = dinv_ref[...]
    h2 = _dense_chain(x_ref[...], dv, p0_ref[0], p1_ref[0],
                      wg_ref[...], bg_ref[...], w1_ref[...], b1_ref[...],
                      w2_ref[...], b2_ref[...], l1g_ref[...], l1b_ref[...],
                      l2g_ref[...], l2b_ref[...])
    pad = pad_ref[...] + jnp.dot(
        h2, lvw_ref[...], preferred_element_type=jnp.float32) + lvb_ref[...]
    out_ref[...] = jnp.concatenate([pad, h2], axis=1)


def _row_spec(w):
    return pl.BlockSpec((_BM, w), lambda i: (i, 0))


def _full_spec(a, b):
    return pl.BlockSpec((a, b), lambda i: (0, 0))


_W_SPECS = [
    _full_spec(D, D), _full_spec(1, D),
    _full_spec(D, FFD), _full_spec(1, FFD),
    _full_spec(FFD, D), _full_spec(1, D),
    _full_spec(1, D), _full_spec(1, D), _full_spec(1, D), _full_spec(1, D),
    _full_spec(D, 4), _full_spec(1, 4),
]

_P0_SPEC = pl.BlockSpec((1, _BM, D), lambda i: (0, i, 0))
_P1_SPEC = pl.BlockSpec((1, _BM, D), lambda i: (1, i, 0))

_layer0 = pl.pallas_call(
    _layer0_body,
    grid=(_G,),
    in_specs=[_row_spec(D), _P0_SPEC, _P1_SPEC, _row_spec(1)]
    + _W_SPECS,
    out_specs=[_row_spec(D), _row_spec(D), _row_spec(4)],
    out_shape=[
        jax.ShapeDtypeStruct((N, D), jnp.float32),
        jax.ShapeDtypeStruct((N, D), jnp.float32),
        jax.ShapeDtypeStruct((N, 4), jnp.float32),
    ],
)

_layerf = pl.pallas_call(
    _layerf_body,
    grid=(_G,),
    in_specs=[_row_spec(D), _P0_SPEC, _P1_SPEC, _row_spec(1),
              _row_spec(4)] + _W_SPECS,
    out_specs=pl.BlockSpec((_BM, 4 + D), lambda i: (i, 0)),
    out_shape=jax.ShapeDtypeStruct((N, 4 + D), jnp.float32),
)


def kernel(x, edge_index, node_indices,
           Wg0, bg0, ln1g0, ln1b0, W10, b10, W20, b20, ln2g0, ln2b0,
           LvW0, Lvb0,
           Wg1, bg1, ln1g1, ln1b1, W11, b11, W21, b21, ln2g1, ln2b1,
           LvW1, Lvb1):
    del node_indices
    e4 = edge_index.reshape(2, NW, NWIN, W, C)

    degp = _deg_kernel(e4)
    da = degp[0, :N].reshape(N, 1)
    db = degp[1, :N].reshape(N, 1)
    dinv, y0 = _prep(x, da, db)

    def lparams(Wg, bg, W1, b1, W2, b2, l1g, l1b, l2g, l2b, LvW, Lvb):
        return (Wg, bg.reshape(1, D), W1, b1.reshape(1, FFD), W2,
                b2.reshape(1, D), l1g.reshape(1, D), l1b.reshape(1, D),
                l2g.reshape(1, D), l2b.reshape(1, D), LvW, Lvb.reshape(1, 4))

    p = _gs_kernel(y0, e4)
    x1, y1, pad1 = _layer0(x, p, p, dinv,
                           *lparams(Wg0, bg0, W10, b10, W20, b20,
                                    ln1g0, ln1b0, ln2g0, ln2b0, LvW0, Lvb0))
    p2 = _gs_kernel(y1, e4)
    return _layerf(x1, p2, p2, dinv, pad1,
                   *lparams(Wg1, bg1, W11, b11, W21, b21,
                            ln1g1, ln1b1, ln2g1, ln2b1, LvW1, Lvb1))


# speedup vs baseline: 1.0317x; 1.0317x over previous
import functools

import jax
import jax.numpy as jnp
from jax import lax
from jax.experimental import pallas as pl
from jax.experimental.pallas import tpu as pltpu
from jax.experimental.pallas import tpu_sc as plsc

N = 10000
E = 320000
D = 128
FFD = 256
NC = 2
NS = 16
NW = NC * NS
EPT = E // NW
C = 80
NCH = EPT // C
W = 25
NWIN = NCH // W
NPAD = 10240

_sc_mesh = plsc.VectorSubcoreMesh(
    core_axis_name="c", subcore_axis_name="s", num_cores=NC, num_subcores=NS)


@functools.partial(
    pl.kernel,
    out_type=jax.ShapeDtypeStruct((NC, NPAD), jnp.float32),
    mesh=_sc_mesh,
    scratch_types=[
        pltpu.VMEM((NCH, C), jnp.int32),
        pltpu.VMEM((C,), jnp.float32),
        pltpu.VMEM((NPAD // NS,), jnp.float32),
        pltpu.VMEM_SHARED((NPAD,), jnp.float32),
        pltpu.SemaphoreType.DMA,
    ],
)
def _deg_kernel(e4_hbm, deg_out, dst_v, ones_v, z_v, deg_sh, isem):
    cid = lax.axis_index("c")
    sid = lax.axis_index("s")
    wid = sid * NC + cid

    for w in range(NWIN):
        pltpu.async_copy(e4_hbm.at[1, wid, w],
                         dst_v.at[pl.ds(w * W, W)], isem)

    def zb(i, carry):
        z_v[pl.ds(i * 16, 16)] = jnp.zeros((16,), jnp.float32)
        return carry

    lax.fori_loop(0, NPAD // NS // 16, zb, 0)
    pltpu.sync_copy(z_v, deg_sh.at[pl.ds(sid * (NPAD // NS), NPAD // NS)])
    for k in range(C // 16):
        ones_v[pl.ds(k * 16, 16)] = jnp.full((16,), 1.0, jnp.float32)
    for w in range(NWIN):
        pltpu.make_async_copy(e4_hbm.at[1, wid, w],
                              dst_v.at[pl.ds(w * W, W)], isem).wait()
    plsc.subcore_barrier()

    def body(j, carry):
        pltpu.sync_copy(ones_v, deg_sh.at[dst_v.at[j]], add=True)
        return carry

    lax.fori_loop(0, NCH, body, 0)
    plsc.subcore_barrier()

    @pl.when(sid == 0)
    def _():
        pltpu.sync_copy(deg_sh, deg_out.at[cid])


def _make_gs(mode):
  @functools.partial(
    pl.kernel,
    out_type=jax.ShapeDtypeStruct((NC, N, D), jnp.float32),
    mesh=_sc_mesh,
    scratch_types=[
        pltpu.VMEM((2, W, C), jnp.int32),
        pltpu.VMEM((2, W, C), jnp.int32),
        pltpu.VMEM((3, C, D), jnp.float32),
        pltpu.VMEM_SHARED((N, D), jnp.float32),
        pltpu.SemaphoreType.DMA,
        pltpu.SemaphoreType.DMA,
        pltpu.SemaphoreType.DMA,
    ],
  )
  def _gs_kernel(y_hbm, e4_hbm, agg_out,
                 src_v, dst_v, rows_v, agg_sh, gsem, ssem, isem):
    cid = lax.axis_index("c")
    sid = lax.axis_index("s")
    wid = sid * NC + cid

    def zbody(r, carry):
        for k in range(D // 16):
            rows_v[0, r, pl.ds(k * 16, 16)] = jnp.zeros((16,), jnp.float32)
        return carry

    lax.fori_loop(0, C, zbody, 0)
    for k in range(8):
        ch = sid + k * NS

        @pl.when(ch < NCH)
        def _():
            pltpu.sync_copy(rows_v.at[0], agg_sh.at[pl.ds(ch * C, C)])

    plsc.subcore_barrier()

    def _drain_scatter():
        pltpu.make_async_copy(rows_v.at[0], agg_sh.at[dst_v.at[0, 0]],
                              ssem).wait()

    pltpu.async_copy(e4_hbm.at[0, wid, 0], src_v.at[0], isem)
    pltpu.async_copy(e4_hbm.at[1, wid, 0], dst_v.at[0], isem)

    def wbody(w, carry):
        wb = lax.rem(w, 2)
        pltpu.make_async_copy(e4_hbm.at[0, wid, w], src_v.at[wb],
                              isem).wait()
        pltpu.make_async_copy(e4_hbm.at[1, wid, w], dst_v.at[wb],
                              isem).wait()
        if mode != "scatter":
            pltpu.async_copy(y_hbm.at[src_v.at[wb, 0]], rows_v.at[0], gsem)
        @pl.when(w + 1 < NWIN)
        def _():
            nwb = lax.rem(w + 1, 2)
            pltpu.async_copy(e4_hbm.at[0, wid, w + 1], src_v.at[nwb], isem)
            pltpu.async_copy(e4_hbm.at[1, wid, w + 1], dst_v.at[nwb], isem)

        def body(j, carry2):
            if mode != "gather":
                @pl.when(j >= 2)
                def _():
                    _drain_scatter()

            if mode != "scatter":
                @pl.when(j + 1 < W)
                def _():
                    pltpu.async_copy(y_hbm.at[src_v.at[wb, j + 1]],
                                     rows_v.at[lax.rem(j + 1, 3)], gsem)

                pltpu.make_async_copy(y_hbm.at[src_v.at[wb, j]],
                                      rows_v.at[lax.rem(j, 3)], gsem).wait()
            if mode != "gather":
                pltpu.async_copy(rows_v.at[lax.rem(j, 3)],
                                 agg_sh.at[dst_v.at[wb, j]], ssem, add=True)
            return carry2

        lax.fori_loop(0, W, body, 0)
        if mode != "gather":
            _drain_scatter()
            _drain_scatter()
        return carry

    lax.fori_loop(0, NWIN, wbody, 0)
    plsc.subcore_barrier()

    for k in range(8):
        ch = sid + k * NS

        @pl.when(ch < NCH)
        def _():
            pltpu.sync_copy(agg_sh.at[pl.ds(ch * C, C)],
                            agg_out.at[cid, pl.ds(ch * C, C)])

  return _gs_kernel


_gs_kernel = _make_gs("full")


_BM = 5000
_G = N // _BM


def _ln(h):
    mu = jnp.mean(h, axis=-1, keepdims=True)
    var = jnp.mean((h - mu) ** 2, axis=-1, keepdims=True)
    return (h - mu) * lax.rsqrt(var + 1e-5)


def _prep_body(x_ref, da_ref, db_ref, dinv_ref, y_ref):
    deg = da_ref[...] + db_ref[...] + 1.0
    dinv = lax.rsqrt(deg)
    dinv_ref[...] = dinv
    y_ref[...] = x_ref[...] * dinv


_prep = pl.pallas_call(
    _prep_body,
    grid=(_G,),
    in_specs=[
        pl.BlockSpec((_BM, D), lambda i: (i, 0)),
        pl.BlockSpec((_BM, 1), lambda i: (i, 0)),
        pl.BlockSpec((_BM, 1), lambda i: (i, 0)),
    ],
    out_specs=[
        pl.BlockSpec((_BM, 1), lambda i: (i, 0)),
        pl.BlockSpec((_BM, D), lambda i: (i, 0)),
    ],
    out_shape=[
        jax.ShapeDtypeStruct((N, 1), jnp.float32),
        jax.ShapeDtypeStruct((N, D), jnp.float32),
    ],
)


def _dense_chain(xb, dv, p0, p1, wg, bg, w1, b1, w2, b2, l1g, l1b, l2g, l2b):
    agg = dv * (p0 + p1) + (dv * dv) * xb
    hg = jnp.dot(agg, wg, preferred_element_type=jnp.float32)
    h = xb + hg + bg
    h = _ln(h) * l1g + l1b
    t = jnp.maximum(
        jnp.dot(h, w1, preferred_element_type=jnp.float32) + b1, 0.0)
    ff = jnp.dot(t, w2, preferred_element_type=jnp.float32) + b2
    return _ln(h + ff) * l2g + l2b


def _layer0_body(x_ref, p0_ref, p1_ref, dinv_ref,
                 wg_ref, bg_ref, w1_ref, b1_ref, w2_ref, b2_ref,
                 l1g_ref, l1b_ref, l2g_ref, l2b_ref, lvw_ref, lvb_ref,
                 xo_ref, yo_ref, pado_ref):
    dv = dinv_ref[...]
    h2 = _dense_chain(x_ref[...], dv, p0_ref[0], p1_ref[0],
                      wg_ref[...], bg_ref[...], w1_ref[...], b1_ref[...],
                      w2_ref[...], b2_ref[...], l1g_ref[...], l1b_ref[...],
                      l2g_ref[...], l2b_ref[...])
    xo_ref[...] = h2
    yo_ref[...] = h2 * dv
    pado_ref[...] = jnp.dot(
        h2, lvw_ref[...], preferred_element_type=jnp.float32) + lvb_ref[...]


def _layerf_body(x_ref, p0_ref, p1_ref, dinv_ref, pad_ref,
                 wg_ref, bg_ref, w1_ref, b1_ref, w2_ref, b2_ref,
                 l1g_ref, l1b_ref, l2g_ref, l2b_ref, lvw_ref, lvb_ref,
                 out_ref):
    dv = dinv_ref[...]
    h2 = _dense_chain(x_ref[...], dv, p0_ref[0], p1_ref[0],
                      wg_ref[...], bg_ref[...], w1_ref[...], b1_ref[...],
                      w2_ref[...], b2_ref[...], l1g_ref[...], l1b_ref[...],
                      l2g_ref[...], l2b_ref[...])
    pad = pad_ref[...] + jnp.dot(
        h2, lvw_ref[...], preferred_element_type=jnp.float32) + lvb_ref[...]
    out_ref[...] = jnp.concatenate([pad, h2], axis=1)


def _row_spec(w):
    return pl.BlockSpec((_BM, w), lambda i: (i, 0))


def _full_spec(a, b):
    return pl.BlockSpec((a, b), lambda i: (0, 0))


_W_SPECS = [
    _full_spec(D, D), _full_spec(1, D),
    _full_spec(D, FFD), _full_spec(1, FFD),
    _full_spec(FFD, D), _full_spec(1, D),
    _full_spec(1, D), _full_spec(1, D), _full_spec(1, D), _full_spec(1, D),
    _full_spec(D, 4), _full_spec(1, 4),
]

_P0_SPEC = pl.BlockSpec((1, _BM, D), lambda i: (0, i, 0))
_P1_SPEC = pl.BlockSpec((1, _BM, D), lambda i: (1, i, 0))

_layer0 = pl.pallas_call(
    _layer0_body,
    grid=(_G,),
    in_specs=[_row_spec(D), _P0_SPEC, _P1_SPEC, _row_spec(1)]
    + _W_SPECS,
    out_specs=[_row_spec(D), _row_spec(D), _row_spec(4)],
    out_shape=[
        jax.ShapeDtypeStruct((N, D), jnp.float32),
        jax.ShapeDtypeStruct((N, D), jnp.float32),
        jax.ShapeDtypeStruct((N, 4), jnp.float32),
    ],
)

_layerf = pl.pallas_call(
    _layerf_body,
    grid=(_G,),
    in_specs=[_row_spec(D), _P0_SPEC, _P1_SPEC, _row_spec(1),
              _row_spec(4)] + _W_SPECS,
    out_specs=pl.BlockSpec((_BM, 4 + D), lambda i: (i, 0)),
    out_shape=jax.ShapeDtypeStruct((N, 4 + D), jnp.float32),
)


def kernel(x, edge_index, node_indices,
           Wg0, bg0, ln1g0, ln1b0, W10, b10, W20, b20, ln2g0, ln2b0,
           LvW0, Lvb0,
           Wg1, bg1, ln1g1, ln1b1, W11, b11, W21, b21, ln2g1, ln2b1,
           LvW1, Lvb1):
    del node_indices
    e4 = edge_index.reshape(2, NW, NWIN, W, C)

    degp = _deg_kernel(e4)
    da = degp[0, :N].reshape(N, 1)
    db = degp[1, :N].reshape(N, 1)
    dinv, y0 = _prep(x, da, db)

    def lparams(Wg, bg, W1, b1, W2, b2, l1g, l1b, l2g, l2b, LvW, Lvb):
        return (Wg, bg.reshape(1, D), W1, b1.reshape(1, FFD), W2,
                b2.reshape(1, D), l1g.reshape(1, D), l1b.reshape(1, D),
                l2g.reshape(1, D), l2b.reshape(1, D), LvW, Lvb.reshape(1, 4))

    p = _gs_kernel(y0, e4)
    x1, y1, pad1 = _layer0(x, p, p, dinv,
                           *lparams(Wg0, bg0, W10, b10, W20, b20,
                                    ln1g0, ln1b0, ln2g0, ln2b0, LvW0, Lvb0))
    p2 = _gs_kernel(y1, e4)
    return _layerf(x1, p2, p2, dinv, pad1,
                   *lparams(Wg1, bg1, W11, b11, W21, b21,
                            ln1g1, ln1b1, ln2g1, ln2b1, LvW1, Lvb1))
